# baseline (speedup 1.0000x reference)
import functools

import jax
import jax.numpy as jnp
from jax import lax
from jax.experimental import pallas as pl
from jax.experimental.pallas import tpu as pltpu
from jax.experimental.pallas import tpu_sc as plsc

_NC = 2
_NS = 16
_NW = _NC * _NS
_CHUNK = 80

_F32 = jnp.float32
_HIGH = jax.lax.Precision.HIGHEST


def _dot(a, b):
    return jnp.dot(a, b, precision=_HIGH, preferred_element_type=_F32)


def _sc_degree(dst, n_nodes):
    e = dst.shape[0]
    per_w = e // _NW
    mesh = plsc.VectorSubcoreMesh(
        core_axis_name="c", subcore_axis_name="s",
        num_cores=_NC, num_subcores=_NS)

    @functools.partial(
        pl.kernel,
        out_type=jax.ShapeDtypeStruct((_NW, n_nodes), _F32),
        mesh=mesh,
        scratch_types=[
            pltpu.VMEM((per_w,), jnp.int32),
            pltpu.VMEM((n_nodes,), _F32),
        ],
        compiler_params=pltpu.CompilerParams(needs_layout_passes=False),
    )
    def k(dst_hbm, out_hbm, dst_v, deg_v):
        wid = lax.axis_index("s") * _NC + lax.axis_index("c")
        pltpu.sync_copy(dst_hbm.at[pl.ds(wid * per_w, per_w)], dst_v)

        def zero_body(i, _):
            deg_v[pl.ds(i * 16, 16)] = jnp.zeros((16,), _F32)
            return 0

        lax.fori_loop(0, n_nodes // 16, zero_body, 0)
        ones = jnp.ones((16,), _F32)

        def body(i, _):
            idx = dst_v[pl.ds(i * 16, 16)]
            plsc.addupdate_scatter(deg_v, [idx], ones)
            return 0

        lax.fori_loop(0, per_w // 16, body, 0)
        pltpu.sync_copy(deg_v, out_hbm.at[wid])

    return k(dst)


def _sc_scatter(src2d, dst3d, g, zeros_tile):
    n, d = g.shape
    per_w = src2d.shape[1]
    chunks_per_w = dst3d.shape[1]
    n_pad = zeros_tile.shape[0] * _NS
    rows_per_tile = n_pad // _NS
    mesh = plsc.VectorSubcoreMesh(
        core_axis_name="c", subcore_axis_name="s",
        num_cores=_NC, num_subcores=_NS)

    assert chunks_per_w % 2 == 1

    @functools.partial(
        pl.kernel,
        out_type=jax.ShapeDtypeStruct((_NC, n_pad, d), _F32),
        mesh=mesh,
        scratch_types=[
            pltpu.VMEM((per_w,), jnp.int32),
            pltpu.VMEM((chunks_per_w, _CHUNK), jnp.int32),
            [pltpu.VMEM((_CHUNK, d), _F32) for _ in range(2)],
            pltpu.VMEM_SHARED((n_pad, d), _F32),
            [pltpu.SemaphoreType.DMA for _ in range(2)],
            [pltpu.SemaphoreType.DMA for _ in range(2)],
        ],
        compiler_params=pltpu.CompilerParams(needs_layout_passes=False),
    )
    def k(src_hbm, dst_hbm, g_hbm, zero_hbm, out_hbm,
          src_v, dst_v, rows, acc, semg, sems):
        cid = lax.axis_index("c")
        sid = lax.axis_index("s")
        wid = sid * _NC + cid
        pltpu.sync_copy(zero_hbm, acc.at[pl.ds(sid * rows_per_tile, rows_per_tile)])
        pltpu.sync_copy(src_hbm.at[wid], src_v)
        pltpu.sync_copy(dst_hbm.at[wid], dst_v)
        plsc.subcore_barrier()

        def gather_start(j, b):
            pltpu.async_copy(
                g_hbm.at[src_v.at[pl.ds(j * _CHUNK, _CHUNK)]], rows[b], semg[b])

        def gather_wait(j, b):
            pltpu.make_async_copy(
                g_hbm.at[src_v.at[pl.ds(j * _CHUNK, _CHUNK)]], rows[b],
                semg[b]).wait()

        def scat_start(j, b):
            pltpu.async_copy(rows[b], acc.at[dst_v.at[j]], sems[b], add=True)

        def scat_wait(j, b):
            pltpu.make_async_copy(rows[b], acc.at[dst_v.at[j]], sems[b]).wait()

        last = chunks_per_w - 1
        gather_start(0, 0)
        gather_wait(0, 0)
        scat_start(0, 0)
        gather_start(1, 1)
        gather_wait(1, 1)
        scat_start(1, 1)
        scat_wait(0, 0)
        gather_start(2, 0)

        def body(jj, _):
            j = 2 * jj
            gather_wait(j, 0)
            scat_start(j, 0)
            scat_wait(j - 1, 1)
            gather_start(j + 1, 1)
            gather_wait(j + 1, 1)
            scat_start(j + 1, 1)
            scat_wait(j, 0)
            gather_start(j + 2, 0)
            return 0

        lax.fori_loop(1, chunks_per_w // 2, body, 0)
        gather_wait(last, 0)
        scat_start(last, 0)
        scat_wait(last - 1, 1)
        scat_wait(last, 0)

        plsc.subcore_barrier()
        pltpu.sync_copy(
            acc.at[pl.ds(sid * rows_per_tile, rows_per_tile)],
            out_hbm.at[cid, pl.ds(sid * rows_per_tile, rows_per_tile)])

    return k(src2d, dst3d, g, zeros_tile)


_ROWS = 2000


def _tc_prep(deg_parts, x):
    n, d = x.shape

    def body(deg_ref, x_ref, dis_ref, g_ref):
        deg = jnp.sum(deg_ref[...], axis=0)
        dis = jnp.where(deg > 0, lax.rsqrt(jnp.maximum(deg, 1e-12)), 0.0)
        dis = dis[:, None]
        dis_ref[...] = dis
        g_ref[...] = dis * x_ref[...]

    return pl.pallas_call(
        body,
        out_shape=[
            jax.ShapeDtypeStruct((n, 1), _F32),
            jax.ShapeDtypeStruct((n, d), _F32),
        ],
    )(deg_parts, x)


def _tc_combine_scale(parts, dis, squared):
    _, n, d = parts.shape
    grid = (n // _ROWS,)

    def body(p_ref, dis_ref, o_ref):
        s = dis_ref[...]
        if squared:
            s = s * s
        o_ref[...] = s * (p_ref[0] + p_ref[1])

    return pl.pallas_call(
        body,
        grid=grid,
        in_specs=[
            pl.BlockSpec((2, _ROWS, d), lambda i: (0, i, 0)),
            pl.BlockSpec((_ROWS, 1), lambda i: (i, 0)),
        ],
        out_specs=pl.BlockSpec((_ROWS, d), lambda i: (i, 0)),
        out_shape=jax.ShapeDtypeStruct((n, d), _F32),
    )(parts, dis)


def _tc_layer(u_parts, v_parts, dis, x, w10, w11, w12, w20, w21, w22):
    n, d = x.shape
    grid = (n // _ROWS,)

    def body(u_ref, v_ref, dis_ref, x_ref,
             w10_ref, w11_ref, w12_ref, w20_ref, w21_ref, w22_ref,
             q0_ref, m1_ref, m2_ref):
        dis_b = dis_ref[...]
        u = dis_b * (u_ref[0] + u_ref[1])
        v = dis_b * (v_ref[0] + v_ref[1])
        h0 = _dot(x_ref[...], w10_ref[...])
        h1 = _dot(u, w11_ref[...])
        h2 = _dot(v, w12_ref[...])
        h = jax.nn.relu(jnp.concatenate([h0, h1, h2], axis=-1))
        q0_ref[...] = _dot(h, w20_ref[...])
        m1_ref[...] = dis_b * _dot(h, w21_ref[...])
        m2_ref[...] = dis_b * _dot(h, w22_ref[...])

    wspec = lambda shape: pl.BlockSpec(shape, lambda i: (0, 0))
    return pl.pallas_call(
        body,
        grid=grid,
        in_specs=[
            pl.BlockSpec((2, _ROWS, d), lambda i: (0, i, 0)),
            pl.BlockSpec((2, _ROWS, d), lambda i: (0, i, 0)),
            pl.BlockSpec((_ROWS, 1), lambda i: (i, 0)),
            pl.BlockSpec((_ROWS, d), lambda i: (i, 0)),
            wspec(w10.shape), wspec(w11.shape), wspec(w12.shape),
            wspec(w20.shape), wspec(w21.shape), wspec(w22.shape),
        ],
        out_specs=[
            pl.BlockSpec((_ROWS, d), lambda i: (i, 0)),
            pl.BlockSpec((_ROWS, d), lambda i: (i, 0)),
            pl.BlockSpec((_ROWS, d), lambda i: (i, 0)),
        ],
        out_shape=[
            jax.ShapeDtypeStruct((n, d), _F32),
            jax.ShapeDtypeStruct((n, d), _F32),
            jax.ShapeDtypeStruct((n, d), _F32),
        ],
    )(u_parts, v_parts, dis, x, w10, w11, w12, w20, w21, w22)


def _tc_final(q0, a_parts, c_parts, dis):
    n, d = q0.shape
    grid = (n // _ROWS,)

    def body(q0_ref, a_ref, c_ref, dis_ref, o_ref):
        dis_b = dis_ref[...]
        q1 = dis_b * (a_ref[0] + a_ref[1])
        q2 = dis_b * (c_ref[0] + c_ref[1])
        o_ref[...] = jnp.concatenate([q0_ref[...], q1, q2], axis=-1)

    return pl.pallas_call(
        body,
        grid=grid,
        in_specs=[
            pl.BlockSpec((_ROWS, d), lambda i: (i, 0)),
            pl.BlockSpec((2, _ROWS, d), lambda i: (0, i, 0)),
            pl.BlockSpec((2, _ROWS, d), lambda i: (0, i, 0)),
            pl.BlockSpec((_ROWS, 1), lambda i: (i, 0)),
        ],
        out_specs=pl.BlockSpec((_ROWS, 3 * d), lambda i: (i, 0)),
        out_shape=jax.ShapeDtypeStruct((n, 3 * d), _F32),
    )(q0, a_parts, c_parts, dis)


def kernel(x, edge_index, W1_0, W1_1, W1_2, W2_0, W2_1, W2_2):
    n, d = x.shape
    e = edge_index.shape[1]
    src = edge_index[0].astype(jnp.int32)
    dst = edge_index[1].astype(jnp.int32)
    chunks_per_w = e // (_NW * _CHUNK)
    src2d = src.reshape(_NW, e // _NW)
    dst3d = dst.reshape(_NW, chunks_per_w, _CHUNK)
    rpt = -(-n // (8 * _NS)) * 8
    zeros_tile = jnp.zeros((rpt, d), _F32)

    deg_parts = _sc_degree(dst, n)
    dis, g = _tc_prep(deg_parts, x)

    u_parts = _sc_scatter(src2d, dst3d, g, zeros_tile)
    g2 = _tc_combine_scale(u_parts, dis, squared=True)
    v_parts = _sc_scatter(src2d, dst3d, g2, zeros_tile)

    q0, m1, m2 = _tc_layer(u_parts, v_parts, dis, x,
                           W1_0, W1_1, W1_2, W2_0, W2_1, W2_2)

    a_parts = _sc_scatter(src2d, dst3d, m1, zeros_tile)
    b_parts = _sc_scatter(src2d, dst3d, m2, zeros_tile)
    t = _tc_combine_scale(b_parts, dis, squared=True)
    c_parts = _sc_scatter(src2d, dst3d, t, zeros_tile)

    return _tc_final(q0, a_parts, c_parts, dis)


# speedup vs baseline: 1.2495x; 1.2495x over previous
import functools

import jax
import jax.numpy as jnp
from jax import lax
from jax.experimental import pallas as pl
from jax.experimental.pallas import tpu as pltpu
from jax.experimental.pallas import tpu_sc as plsc

_NC = 2
_NS = 16
_NW = _NC * _NS
_CHUNK = 80

_F32 = jnp.float32
_HIGH = jax.lax.Precision.HIGHEST


def _dot(a, b):
    return jnp.dot(a, b, precision=_HIGH, preferred_element_type=_F32)


def _sc_degree(dst, n_nodes):
    e = dst.shape[0]
    per_w = e // _NW
    mesh = plsc.VectorSubcoreMesh(
        core_axis_name="c", subcore_axis_name="s",
        num_cores=_NC, num_subcores=_NS)

    @functools.partial(
        pl.kernel,
        out_type=jax.ShapeDtypeStruct((_NW, n_nodes), _F32),
        mesh=mesh,
        scratch_types=[
            pltpu.VMEM((per_w,), jnp.int32),
            pltpu.VMEM((n_nodes,), _F32),
        ],
        compiler_params=pltpu.CompilerParams(needs_layout_passes=False),
    )
    def k(dst_hbm, out_hbm, dst_v, deg_v):
        wid = lax.axis_index("s") * _NC + lax.axis_index("c")
        pltpu.sync_copy(dst_hbm.at[pl.ds(wid * per_w, per_w)], dst_v)

        def zero_body(i, _):
            deg_v[pl.ds(i * 16, 16)] = jnp.zeros((16,), _F32)
            return 0

        lax.fori_loop(0, n_nodes // 16, zero_body, 0)
        ones = jnp.ones((16,), _F32)

        def body(i, _):
            idx = dst_v[pl.ds(i * 16, 16)]
            plsc.addupdate_scatter(deg_v, [idx], ones)
            return 0

        lax.fori_loop(0, per_w // 16, body, 0)
        pltpu.sync_copy(deg_v, out_hbm.at[wid])

    return k(dst)


def _sc_scatter(src2d, dst3d, gs, zeros_tile):
    n_phase = len(gs)
    n, d = gs[0].shape
    per_w = src2d.shape[1]
    chunks_per_w = dst3d.shape[1]
    n_pad = zeros_tile.shape[0] * _NS
    rows_per_tile = n_pad // _NS
    mesh = plsc.VectorSubcoreMesh(
        core_axis_name="c", subcore_axis_name="s",
        num_cores=_NC, num_subcores=_NS)

    assert chunks_per_w % 2 == 1

    @functools.partial(
        pl.kernel,
        out_type=jax.ShapeDtypeStruct((n_phase, _NC, n_pad, d), _F32),
        mesh=mesh,
        scratch_types=[
            pltpu.VMEM((per_w,), jnp.int32),
            pltpu.VMEM((chunks_per_w, _CHUNK), jnp.int32),
            [pltpu.VMEM((_CHUNK, d), _F32) for _ in range(2)],
            pltpu.VMEM_SHARED((n_pad, d), _F32),
            [pltpu.SemaphoreType.DMA for _ in range(2)],
        ],
        compiler_params=pltpu.CompilerParams(needs_layout_passes=False),
    )
    def k(src_hbm, dst_hbm, *refs):
        g_hbms = refs[:n_phase]
        zero_hbm, out_hbm, src_v, dst_v, rows, acc, semg = refs[n_phase:]
        cid = lax.axis_index("c")
        sid = lax.axis_index("s")
        wid = sid * _NC + cid
        my_rows = pl.ds(sid * rows_per_tile, rows_per_tile)
        pltpu.sync_copy(zero_hbm, acc.at[my_rows])
        pltpu.sync_copy(src_hbm.at[wid], src_v)
        pltpu.sync_copy(dst_hbm.at[wid], dst_v)
        plsc.subcore_barrier()

        def run_phase(g_hbm):
            def gather_start(j, b):
                pltpu.async_copy(
                    g_hbm.at[src_v.at[pl.ds(j * _CHUNK, _CHUNK)]], rows[b],
                    semg[b])

            def gather_wait(j, b):
                pltpu.make_async_copy(
                    g_hbm.at[src_v.at[pl.ds(j * _CHUNK, _CHUNK)]], rows[b],
                    semg[b]).wait()

            def scat(j, b):
                pltpu.sync_copy(rows[b], acc.at[dst_v.at[j]], add=True)

            last = chunks_per_w - 1
            gather_start(0, 0)

            def body(jj, _):
                j = 2 * jj
                gather_start(j + 1, 1)
                gather_wait(j, 0)
                scat(j, 0)
                gather_start(j + 2, 0)
                gather_wait(j + 1, 1)
                scat(j + 1, 1)
                return 0

            lax.fori_loop(0, chunks_per_w // 2, body, 0)
            gather_wait(last, 0)
            scat(last, 0)

        for p, g_hbm in enumerate(g_hbms):
            if p > 0:
                pltpu.sync_copy(zero_hbm, acc.at[my_rows])
                plsc.subcore_barrier()
            run_phase(g_hbm)
            plsc.subcore_barrier()
            pltpu.sync_copy(acc.at[my_rows], out_hbm.at[p, cid, my_rows])

    return k(src2d, dst3d, *gs, zeros_tile)


_ROWS = 2000


def _tc_prep(deg_parts, x):
    n, d = x.shape

    def body(deg_ref, x_ref, dis_ref, g_ref):
        deg = jnp.sum(deg_ref[...], axis=0)
        dis = jnp.where(deg > 0, lax.rsqrt(jnp.maximum(deg, 1e-12)), 0.0)
        dis = dis[:, None]
        dis_ref[...] = dis
        g_ref[...] = dis * x_ref[...]

    return pl.pallas_call(
        body,
        out_shape=[
            jax.ShapeDtypeStruct((n, 1), _F32),
            jax.ShapeDtypeStruct((n, d), _F32),
        ],
    )(deg_parts, x)


def _tc_combine_scale(parts, dis, squared):
    _, n, d = parts.shape
    grid = (n // _ROWS,)

    def body(p_ref, dis_ref, o_ref):
        s = dis_ref[...]
        if squared:
            s = s * s
        o_ref[...] = s * (p_ref[0] + p_ref[1])

    return pl.pallas_call(
        body,
        grid=grid,
        in_specs=[
            pl.BlockSpec((2, _ROWS, d), lambda i: (0, i, 0)),
            pl.BlockSpec((_ROWS, 1), lambda i: (i, 0)),
        ],
        out_specs=pl.BlockSpec((_ROWS, d), lambda i: (i, 0)),
        out_shape=jax.ShapeDtypeStruct((n, d), _F32),
    )(parts, dis)


def _tc_layer(u_parts, v_parts, dis, x, w10, w11, w12, w20, w21, w22):
    n, d = x.shape
    grid = (n // _ROWS,)

    def body(u_ref, v_ref, dis_ref, x_ref,
             w10_ref, w11_ref, w12_ref, w20_ref, w21_ref, w22_ref,
             q0_ref, m1_ref, m2_ref):
        dis_b = dis_ref[...]
        u = dis_b * (u_ref[0] + u_ref[1])
        v = dis_b * (v_ref[0] + v_ref[1])
        h0 = _dot(x_ref[...], w10_ref[...])
        h1 = _dot(u, w11_ref[...])
        h2 = _dot(v, w12_ref[...])
        h = jax.nn.relu(jnp.concatenate([h0, h1, h2], axis=-1))
        q0_ref[...] = _dot(h, w20_ref[...])
        m1_ref[...] = dis_b * _dot(h, w21_ref[...])
        m2_ref[...] = dis_b * _dot(h, w22_ref[...])

    wspec = lambda shape: pl.BlockSpec(shape, lambda i: (0, 0))
    return pl.pallas_call(
        body,
        grid=grid,
        in_specs=[
            pl.BlockSpec((2, _ROWS, d), lambda i: (0, i, 0)),
            pl.BlockSpec((2, _ROWS, d), lambda i: (0, i, 0)),
            pl.BlockSpec((_ROWS, 1), lambda i: (i, 0)),
            pl.BlockSpec((_ROWS, d), lambda i: (i, 0)),
            wspec(w10.shape), wspec(w11.shape), wspec(w12.shape),
            wspec(w20.shape), wspec(w21.shape), wspec(w22.shape),
        ],
        out_specs=[
            pl.BlockSpec((_ROWS, d), lambda i: (i, 0)),
            pl.BlockSpec((_ROWS, d), lambda i: (i, 0)),
            pl.BlockSpec((_ROWS, d), lambda i: (i, 0)),
        ],
        out_shape=[
            jax.ShapeDtypeStruct((n, d), _F32),
            jax.ShapeDtypeStruct((n, d), _F32),
            jax.ShapeDtypeStruct((n, d), _F32),
        ],
    )(u_parts, v_parts, dis, x, w10, w11, w12, w20, w21, w22)


def _tc_final(q0, a_parts, c_parts, dis):
    n, d = q0.shape
    grid = (n // _ROWS,)

    def body(q0_ref, a_ref, c_ref, dis_ref, o_ref):
        dis_b = dis_ref[...]
        q1 = dis_b * (a_ref[0] + a_ref[1])
        q2 = dis_b * (c_ref[0] + c_ref[1])
        o_ref[...] = jnp.concatenate([q0_ref[...], q1, q2], axis=-1)

    return pl.pallas_call(
        body,
        grid=grid,
        in_specs=[
            pl.BlockSpec((_ROWS, d), lambda i: (i, 0)),
            pl.BlockSpec((2, _ROWS, d), lambda i: (0, i, 0)),
            pl.BlockSpec((2, _ROWS, d), lambda i: (0, i, 0)),
            pl.BlockSpec((_ROWS, 1), lambda i: (i, 0)),
        ],
        out_specs=pl.BlockSpec((_ROWS, 3 * d), lambda i: (i, 0)),
        out_shape=jax.ShapeDtypeStruct((n, 3 * d), _F32),
    )(q0, a_parts, c_parts, dis)


def kernel(x, edge_index, W1_0, W1_1, W1_2, W2_0, W2_1, W2_2):
    n, d = x.shape
    e = edge_index.shape[1]
    src = edge_index[0].astype(jnp.int32)
    dst = edge_index[1].astype(jnp.int32)
    chunks_per_w = e // (_NW * _CHUNK)
    src2d = src.reshape(_NW, e // _NW)
    dst3d = dst.reshape(_NW, chunks_per_w, _CHUNK)
    rpt = -(-n // (8 * _NS)) * 8
    zeros_tile = jnp.zeros((rpt, d), _F32)

    deg_parts = _sc_degree(dst, n)
    dis, g = _tc_prep(deg_parts, x)

    u_parts = _sc_scatter(src2d, dst3d, [g], zeros_tile)[0]
    g2 = _tc_combine_scale(u_parts, dis, squared=True)
    v_parts = _sc_scatter(src2d, dst3d, [g2], zeros_tile)[0]

    q0, m1, m2 = _tc_layer(u_parts, v_parts, dis, x,
                           W1_0, W1_1, W1_2, W2_0, W2_1, W2_2)

    ab_parts = _sc_scatter(src2d, dst3d, [m1, m2], zeros_tile)
    a_parts, b_parts = ab_parts[0], ab_parts[1]
    t = _tc_combine_scale(b_parts, dis, squared=True)
    c_parts = _sc_scatter(src2d, dst3d, [t], zeros_tile)[0]

    return _tc_final(q0, a_parts, c_parts, dis)


# speedup vs baseline: 1.2601x; 1.0085x over previous
import functools

import jax
import jax.numpy as jnp
from jax import lax
from jax.experimental import pallas as pl
from jax.experimental.pallas import tpu as pltpu
from jax.experimental.pallas import tpu_sc as plsc

_NC = 2
_NS = 16
_NW = _NC * _NS
_CHUNK = 80

_F32 = jnp.float32
_HIGH = jax.lax.Precision.HIGHEST


def _dot(a, b):
    return jnp.dot(a, b, precision=_HIGH, preferred_element_type=_F32)


def _sc_degree(ei2, n_nodes):
    per_w = ei2.shape[2]
    mesh = plsc.VectorSubcoreMesh(
        core_axis_name="c", subcore_axis_name="s",
        num_cores=_NC, num_subcores=_NS)

    @functools.partial(
        pl.kernel,
        out_type=jax.ShapeDtypeStruct((_NW, n_nodes), _F32),
        mesh=mesh,
        scratch_types=[
            pltpu.VMEM((per_w,), jnp.int32),
            pltpu.VMEM((n_nodes,), _F32),
        ],
        compiler_params=pltpu.CompilerParams(needs_layout_passes=False),
    )
    def k(ei_hbm, out_hbm, dst_v, deg_v):
        wid = lax.axis_index("s") * _NC + lax.axis_index("c")
        pltpu.sync_copy(ei_hbm.at[1, wid], dst_v)

        def zero_body(i, _):
            deg_v[pl.ds(i * 16, 16)] = jnp.zeros((16,), _F32)
            return 0

        lax.fori_loop(0, n_nodes // 16, zero_body, 0)
        ones = jnp.ones((16,), _F32)

        def body(i, _):
            idx = dst_v[pl.ds(i * 16, 16)]
            plsc.addupdate_scatter(deg_v, [idx], ones)
            return 0

        lax.fori_loop(0, per_w // 16, body, 0)
        pltpu.sync_copy(deg_v, out_hbm.at[wid])

    return k(ei2)


def _sc_scatter(ei2, ei4, gs, zeros_tile):
    n_phase = len(gs)
    n, d = gs[0].shape
    per_w = ei2.shape[2]
    chunks_per_w = ei4.shape[2]
    n_pad = zeros_tile.shape[0] * _NS
    rows_per_tile = n_pad // _NS
    mesh = plsc.VectorSubcoreMesh(
        core_axis_name="c", subcore_axis_name="s",
        num_cores=_NC, num_subcores=_NS)

    assert chunks_per_w % 2 == 1

    @functools.partial(
        pl.kernel,
        out_type=jax.ShapeDtypeStruct((n_phase, _NC, n_pad, d), _F32),
        mesh=mesh,
        scratch_types=[
            pltpu.VMEM((per_w,), jnp.int32),
            pltpu.VMEM((chunks_per_w, _CHUNK), jnp.int32),
            [pltpu.VMEM((_CHUNK, d), _F32) for _ in range(2)],
            pltpu.VMEM_SHARED((n_pad, d), _F32),
            [pltpu.SemaphoreType.DMA for _ in range(2)],
        ],
        compiler_params=pltpu.CompilerParams(needs_layout_passes=False),
    )
    def k(ei2_hbm, ei4_hbm, *refs):
        g_hbms = refs[:n_phase]
        zero_hbm, out_hbm, src_v, dst_v, rows, acc, semg = refs[n_phase:]
        cid = lax.axis_index("c")
        sid = lax.axis_index("s")
        wid = sid * _NC + cid
        my_rows = pl.ds(sid * rows_per_tile, rows_per_tile)
        pltpu.sync_copy(zero_hbm, acc.at[my_rows])
        pltpu.sync_copy(ei2_hbm.at[0, wid], src_v)
        pltpu.sync_copy(ei4_hbm.at[1, wid], dst_v)
        plsc.subcore_barrier()

        def run_phase(g_hbm):
            def gather_start(j, b):
                pltpu.async_copy(
                    g_hbm.at[src_v.at[pl.ds(j * _CHUNK, _CHUNK)]], rows[b],
                    semg[b])

            def gather_wait(j, b):
                pltpu.make_async_copy(
                    g_hbm.at[src_v.at[pl.ds(j * _CHUNK, _CHUNK)]], rows[b],
                    semg[b]).wait()

            def scat(j, b):
                pltpu.sync_copy(rows[b], acc.at[dst_v.at[j]], add=True)

            last = chunks_per_w - 1
            gather_start(0, 0)

            def body(jj, _):
                j = 2 * jj
                gather_start(j + 1, 1)
                gather_wait(j, 0)
                scat(j, 0)
                gather_start(j + 2, 0)
                gather_wait(j + 1, 1)
                scat(j + 1, 1)
                return 0

            lax.fori_loop(0, chunks_per_w // 2, body, 0)
            gather_wait(last, 0)
            scat(last, 0)

        for p, g_hbm in enumerate(g_hbms):
            if p > 0:
                pltpu.sync_copy(zero_hbm, acc.at[my_rows])
                plsc.subcore_barrier()
            run_phase(g_hbm)
            plsc.subcore_barrier()
            pltpu.sync_copy(acc.at[my_rows], out_hbm.at[p, cid, my_rows])

    return k(ei2, ei4, *gs, zeros_tile)


_ROWS = 2000


def _tc_prep(deg_parts, x):
    n, d = x.shape

    def body(deg_ref, x_ref, dis_ref, g_ref):
        deg = jnp.sum(deg_ref[...], axis=0)
        dis = jnp.where(deg > 0, lax.rsqrt(jnp.maximum(deg, 1e-12)), 0.0)
        dis = dis[:, None]
        dis_ref[...] = dis
        g_ref[...] = dis * x_ref[...]

    return pl.pallas_call(
        body,
        out_shape=[
            jax.ShapeDtypeStruct((n, 1), _F32),
            jax.ShapeDtypeStruct((n, d), _F32),
        ],
    )(deg_parts, x)


def _tc_combine_scale(parts, dis, squared):
    _, n, d = parts.shape
    grid = (n // _ROWS,)

    def body(p_ref, dis_ref, o_ref):
        s = dis_ref[...]
        if squared:
            s = s * s
        o_ref[...] = s * (p_ref[0] + p_ref[1])

    return pl.pallas_call(
        body,
        grid=grid,
        in_specs=[
            pl.BlockSpec((2, _ROWS, d), lambda i: (0, i, 0)),
            pl.BlockSpec((_ROWS, 1), lambda i: (i, 0)),
        ],
        out_specs=pl.BlockSpec((_ROWS, d), lambda i: (i, 0)),
        out_shape=jax.ShapeDtypeStruct((n, d), _F32),
    )(parts, dis)


def _tc_layer_a(u_parts, v_parts, dis, x, w10, w11, w12, w21):
    n, d = x.shape
    grid = (n // _ROWS,)

    def body(u_ref, v_ref, dis_ref, x_ref,
             w10_ref, w11_ref, w12_ref, w21_ref, h_ref, m1_ref):
        dis_b = dis_ref[...]
        u = dis_b * (u_ref[0] + u_ref[1])
        v = dis_b * (v_ref[0] + v_ref[1])
        h0 = _dot(x_ref[...], w10_ref[...])
        h1 = _dot(u, w11_ref[...])
        h2 = _dot(v, w12_ref[...])
        h = jax.nn.relu(jnp.concatenate([h0, h1, h2], axis=-1))
        h_ref[...] = h
        m1_ref[...] = dis_b * _dot(h, w21_ref[...])

    wspec = lambda shape: pl.BlockSpec(shape, lambda i: (0, 0))
    return pl.pallas_call(
        body,
        grid=grid,
        in_specs=[
            pl.BlockSpec((2, _ROWS, d), lambda i: (0, i, 0)),
            pl.BlockSpec((2, _ROWS, d), lambda i: (0, i, 0)),
            pl.BlockSpec((_ROWS, 1), lambda i: (i, 0)),
            pl.BlockSpec((_ROWS, d), lambda i: (i, 0)),
            wspec(w10.shape), wspec(w11.shape), wspec(w12.shape),
            wspec(w21.shape),
        ],
        out_specs=[
            pl.BlockSpec((_ROWS, 3 * d), lambda i: (i, 0)),
            pl.BlockSpec((_ROWS, d), lambda i: (i, 0)),
        ],
        out_shape=[
            jax.ShapeDtypeStruct((n, 3 * d), _F32),
            jax.ShapeDtypeStruct((n, d), _F32),
        ],
    )(u_parts, v_parts, dis, x, w10, w11, w12, w21)


def _tc_layer_b(h, dis, w20, w22):
    n, d3 = h.shape
    d = d3 // 3
    grid = (n // _ROWS,)

    def body(h_ref, dis_ref, w20_ref, w22_ref, q0_ref, m2_ref):
        h = h_ref[...]
        q0_ref[...] = _dot(h, w20_ref[...])
        m2_ref[...] = dis_ref[...] * _dot(h, w22_ref[...])

    wspec = lambda shape: pl.BlockSpec(shape, lambda i: (0, 0))
    return pl.pallas_call(
        body,
        grid=grid,
        in_specs=[
            pl.BlockSpec((_ROWS, d3), lambda i: (i, 0)),
            pl.BlockSpec((_ROWS, 1), lambda i: (i, 0)),
            wspec(w20.shape), wspec(w22.shape),
        ],
        out_specs=[
            pl.BlockSpec((_ROWS, d), lambda i: (i, 0)),
            pl.BlockSpec((_ROWS, d), lambda i: (i, 0)),
        ],
        out_shape=[
            jax.ShapeDtypeStruct((n, d), _F32),
            jax.ShapeDtypeStruct((n, d), _F32),
        ],
    )(h, dis, w20, w22)


def _tc_final(q0, a_parts, c_parts, dis):
    n, d = q0.shape
    grid = (n // _ROWS,)

    def body(q0_ref, a_ref, c_ref, dis_ref, o_ref):
        dis_b = dis_ref[...]
        q1 = dis_b * (a_ref[0] + a_ref[1])
        q2 = dis_b * (c_ref[0] + c_ref[1])
        o_ref[...] = jnp.concatenate([q0_ref[...], q1, q2], axis=-1)

    return pl.pallas_call(
        body,
        grid=grid,
        in_specs=[
            pl.BlockSpec((_ROWS, d), lambda i: (i, 0)),
            pl.BlockSpec((2, _ROWS, d), lambda i: (0, i, 0)),
            pl.BlockSpec((2, _ROWS, d), lambda i: (0, i, 0)),
            pl.BlockSpec((_ROWS, 1), lambda i: (i, 0)),
        ],
        out_specs=pl.BlockSpec((_ROWS, 3 * d), lambda i: (i, 0)),
        out_shape=jax.ShapeDtypeStruct((n, 3 * d), _F32),
    )(q0, a_parts, c_parts, dis)


def kernel(x, edge_index, W1_0, W1_1, W1_2, W2_0, W2_1, W2_2):
    n, d = x.shape
    e = edge_index.shape[1]
    ei = edge_index.astype(jnp.int32)
    chunks_per_w = e // (_NW * _CHUNK)
    ei2 = ei.reshape(2, _NW, e // _NW)
    ei4 = ei.reshape(2, _NW, chunks_per_w, _CHUNK)
    rpt = -(-n // (8 * _NS)) * 8
    zeros_tile = jnp.zeros((rpt, d), _F32)

    deg_parts = _sc_degree(ei2, n)
    dis, g = _tc_prep(deg_parts, x)

    u_parts = _sc_scatter(ei2, ei4, [g], zeros_tile)[0]
    g2 = _tc_combine_scale(u_parts, dis, squared=True)
    v_parts = _sc_scatter(ei2, ei4, [g2], zeros_tile)[0]

    h, m1 = _tc_layer_a(u_parts, v_parts, dis, x, W1_0, W1_1, W1_2, W2_1)
    a_parts = _sc_scatter(ei2, ei4, [m1], zeros_tile)[0]
    q0, m2 = _tc_layer_b(h, dis, W2_0, W2_2)

    b_parts = _sc_scatter(ei2, ei4, [m2], zeros_tile)[0]
    t = _tc_combine_scale(b_parts, dis, squared=True)
    c_parts = _sc_scatter(ei2, ei4, [t], zeros_tile)[0]

    return _tc_final(q0, a_parts, c_parts, dis)


# speedup vs baseline: 1.2688x; 1.0069x over previous
import functools

import jax
import jax.numpy as jnp
from jax import lax
from jax.experimental import pallas as pl
from jax.experimental.pallas import tpu as pltpu
from jax.experimental.pallas import tpu_sc as plsc

_NC = 2
_NS = 16
_NW = _NC * _NS
_CHUNK = 80

_F32 = jnp.float32
_HIGH = jax.lax.Precision.HIGHEST


def _dot(a, b):
    return jnp.dot(a, b, precision=_HIGH, preferred_element_type=_F32)


def _sc_degree(ei2, n_nodes):
    per_w = ei2.shape[2]
    mesh = plsc.VectorSubcoreMesh(
        core_axis_name="c", subcore_axis_name="s",
        num_cores=_NC, num_subcores=_NS)

    @functools.partial(
        pl.kernel,
        out_type=jax.ShapeDtypeStruct((_NW, n_nodes), _F32),
        mesh=mesh,
        scratch_types=[
            pltpu.VMEM((per_w,), jnp.int32),
            pltpu.VMEM((n_nodes,), _F32),
        ],
        compiler_params=pltpu.CompilerParams(needs_layout_passes=False),
    )
    def k(ei_hbm, out_hbm, dst_v, deg_v):
        wid = lax.axis_index("s") * _NC + lax.axis_index("c")
        pltpu.sync_copy(ei_hbm.at[1, wid], dst_v)

        def zero_body(i, _):
            deg_v[pl.ds(i * 16, 16)] = jnp.zeros((16,), _F32)
            return 0

        lax.fori_loop(0, n_nodes // 16, zero_body, 0)
        ones = jnp.ones((16,), _F32)

        def body(i, _):
            idx = dst_v[pl.ds(i * 16, 16)]
            plsc.addupdate_scatter(deg_v, [idx], ones)
            return 0

        lax.fori_loop(0, per_w // 16, body, 0)
        pltpu.sync_copy(deg_v, out_hbm.at[wid])

    return k(ei2)


def _sc_scatter(ei2, ei4, gs, zeros_tile):
    n_phase = len(gs)
    n, d = gs[0].shape
    per_w = ei2.shape[2]
    chunks_per_w = ei4.shape[2]
    n_pad = zeros_tile.shape[0] * _NS
    rows_per_tile = n_pad // _NS
    mesh = plsc.VectorSubcoreMesh(
        core_axis_name="c", subcore_axis_name="s",
        num_cores=_NC, num_subcores=_NS)

    assert chunks_per_w % 2 == 1

    @functools.partial(
        pl.kernel,
        out_type=jax.ShapeDtypeStruct((n_phase, _NC, n_pad, d), _F32),
        mesh=mesh,
        scratch_types=[
            pltpu.VMEM((per_w,), jnp.int32),
            pltpu.VMEM((chunks_per_w, _CHUNK), jnp.int32),
            [pltpu.VMEM((_CHUNK, d), _F32) for _ in range(2)],
            pltpu.VMEM_SHARED((n_pad, d), _F32),
            [pltpu.SemaphoreType.DMA for _ in range(2)],
        ],
        compiler_params=pltpu.CompilerParams(needs_layout_passes=False),
    )
    def k(ei_hbm, ei4_hbm, *refs):
        g_hbms = refs[:n_phase]
        zero_hbm, out_hbm, src_v, dst_v, rows, acc, semg = refs[n_phase:]
        cid = lax.axis_index("c")
        sid = lax.axis_index("s")
        wid = sid * _NC + cid
        my_rows = pl.ds(sid * rows_per_tile, rows_per_tile)
        pltpu.sync_copy(zero_hbm, acc.at[my_rows])
        pltpu.sync_copy(ei_hbm.at[0, wid], src_v)
        pltpu.sync_copy(ei4_hbm.at[1, wid], dst_v)
        plsc.subcore_barrier()

        def run_phase(g_hbm):
            def gather_start(j, b):
                pltpu.async_copy(
                    g_hbm.at[src_v.at[pl.ds(j * _CHUNK, _CHUNK)]], rows[b],
                    semg[b])

            def gather_wait(j, b):
                pltpu.make_async_copy(
                    g_hbm.at[src_v.at[pl.ds(j * _CHUNK, _CHUNK)]], rows[b],
                    semg[b]).wait()

            def scat(j, b):
                pltpu.sync_copy(rows[b], acc.at[dst_v.at[j]], add=True)

            last = chunks_per_w - 1
            gather_start(0, 0)

            def body(jj, _):
                j = 2 * jj
                gather_start(j + 1, 1)
                gather_wait(j, 0)
                scat(j, 0)
                gather_start(j + 2, 0)
                gather_wait(j + 1, 1)
                scat(j + 1, 1)
                return 0

            lax.fori_loop(0, chunks_per_w // 2, body, 0)
            gather_wait(last, 0)
            scat(last, 0)

        for p, g_hbm in enumerate(g_hbms):
            if p > 0:
                pltpu.sync_copy(zero_hbm, acc.at[my_rows])
                plsc.subcore_barrier()
            run_phase(g_hbm)
            plsc.subcore_barrier()
            pltpu.sync_copy(acc.at[my_rows], out_hbm.at[p, cid, my_rows])

    return k(ei2, ei4, *gs, zeros_tile)


_ROWS = 2000


def _tc_prep(deg_parts, x):
    n, d = x.shape

    def body(deg_ref, x_ref, dis_ref, g_ref):
        deg = jnp.sum(deg_ref[...], axis=0)
        dis = jnp.where(deg > 0, lax.rsqrt(jnp.maximum(deg, 1e-12)), 0.0)
        dis = dis[:, None]
        dis_ref[...] = dis
        g_ref[...] = dis * x_ref[...]

    return pl.pallas_call(
        body,
        out_shape=[
            jax.ShapeDtypeStruct((n, 1), _F32),
            jax.ShapeDtypeStruct((n, d), _F32),
        ],
    )(deg_parts, x)


def _tc_combine_scale(parts, dis, squared):
    _, n, d = parts.shape
    grid = (n // _ROWS,)

    def body(p_ref, dis_ref, o_ref):
        s = dis_ref[...]
        if squared:
            s = s * s
        o_ref[...] = s * (p_ref[0] + p_ref[1])

    return pl.pallas_call(
        body,
        grid=grid,
        in_specs=[
            pl.BlockSpec((2, _ROWS, d), lambda i: (0, i, 0)),
            pl.BlockSpec((_ROWS, 1), lambda i: (i, 0)),
        ],
        out_specs=pl.BlockSpec((_ROWS, d), lambda i: (i, 0)),
        out_shape=jax.ShapeDtypeStruct((n, d), _F32),
    )(parts, dis)


def _tc_layer(u_parts, v_parts, dis, x, w10, w11, w12, w20, w21, w22):
    n, d = x.shape
    grid = (n // _ROWS,)

    def body(u_ref, v_ref, dis_ref, x_ref,
             w10_ref, w11_ref, w12_ref, w20_ref, w21_ref, w22_ref,
             q0_ref, m1_ref, m2_ref):
        dis_b = dis_ref[...]
        u = dis_b * (u_ref[0] + u_ref[1])
        v = dis_b * (v_ref[0] + v_ref[1])
        h0 = _dot(x_ref[...], w10_ref[...])
        h1 = _dot(u, w11_ref[...])
        h2 = _dot(v, w12_ref[...])
        h = jax.nn.relu(jnp.concatenate([h0, h1, h2], axis=-1))
        q0_ref[...] = _dot(h, w20_ref[...])
        m1_ref[...] = dis_b * _dot(h, w21_ref[...])
        m2_ref[...] = dis_b * _dot(h, w22_ref[...])

    wspec = lambda shape: pl.BlockSpec(shape, lambda i: (0, 0))
    return pl.pallas_call(
        body,
        grid=grid,
        in_specs=[
            pl.BlockSpec((2, _ROWS, d), lambda i: (0, i, 0)),
            pl.BlockSpec((2, _ROWS, d), lambda i: (0, i, 0)),
            pl.BlockSpec((_ROWS, 1), lambda i: (i, 0)),
            pl.BlockSpec((_ROWS, d), lambda i: (i, 0)),
            wspec(w10.shape), wspec(w11.shape), wspec(w12.shape),
            wspec(w20.shape), wspec(w21.shape), wspec(w22.shape),
        ],
        out_specs=[
            pl.BlockSpec((_ROWS, d), lambda i: (i, 0)),
            pl.BlockSpec((_ROWS, d), lambda i: (i, 0)),
            pl.BlockSpec((_ROWS, d), lambda i: (i, 0)),
        ],
        out_shape=[
            jax.ShapeDtypeStruct((n, d), _F32),
            jax.ShapeDtypeStruct((n, d), _F32),
            jax.ShapeDtypeStruct((n, d), _F32),
        ],
    )(u_parts, v_parts, dis, x, w10, w11, w12, w20, w21, w22)


def _tc_final(q0, a_parts, c_parts, dis):
    n, d = q0.shape
    grid = (n // _ROWS,)

    def body(q0_ref, a_ref, c_ref, dis_ref, o_ref):
        dis_b = dis_ref[...]
        q1 = dis_b * (a_ref[0] + a_ref[1])
        q2 = dis_b * (c_ref[0] + c_ref[1])
        o_ref[...] = jnp.concatenate([q0_ref[...], q1, q2], axis=-1)

    return pl.pallas_call(
        body,
        grid=grid,
        in_specs=[
            pl.BlockSpec((_ROWS, d), lambda i: (i, 0)),
            pl.BlockSpec((2, _ROWS, d), lambda i: (0, i, 0)),
            pl.BlockSpec((2, _ROWS, d), lambda i: (0, i, 0)),
            pl.BlockSpec((_ROWS, 1), lambda i: (i, 0)),
        ],
        out_specs=pl.BlockSpec((_ROWS, 3 * d), lambda i: (i, 0)),
        out_shape=jax.ShapeDtypeStruct((n, 3 * d), _F32),
    )(q0, a_parts, c_parts, dis)


def kernel(x, edge_index, W1_0, W1_1, W1_2, W2_0, W2_1, W2_2):
    n, d = x.shape
    e = edge_index.shape[1]
    ei = edge_index.astype(jnp.int32)
    chunks_per_w = e // (_NW * _CHUNK)
    ei2 = ei.reshape(2, _NW, e // _NW)
    ei4 = ei.reshape(2, _NW, chunks_per_w, _CHUNK)
    rpt = -(-n // (8 * _NS)) * 8
    zeros_tile = jnp.zeros((rpt, d), _F32)

    deg_parts = _sc_degree(ei2, n)
    dis, g = _tc_prep(deg_parts, x)

    u_parts = _sc_scatter(ei2, ei4, [g], zeros_tile)[0]
    g2 = _tc_combine_scale(u_parts, dis, squared=True)
    v_parts = _sc_scatter(ei2, ei4, [g2], zeros_tile)[0]

    q0, m1, m2 = _tc_layer(u_parts, v_parts, dis, x,
                           W1_0, W1_1, W1_2, W2_0, W2_1, W2_2)

    a_parts = _sc_scatter(ei2, ei4, [m1], zeros_tile)[0]
    b_parts = _sc_scatter(ei2, ei4, [m2], zeros_tile)[0]
    t = _tc_combine_scale(b_parts, dis, squared=True)
    c_parts = _sc_scatter(ei2, ei4, [t], zeros_tile)[0]

    return _tc_final(q0, a_parts, c_parts, dis)


# speedup vs baseline: 1.3378x; 1.0544x over previous
import functools

import jax
import jax.numpy as jnp
from jax import lax
from jax.experimental import pallas as pl
from jax.experimental.pallas import tpu as pltpu
from jax.experimental.pallas import tpu_sc as plsc

_NC = 2
_NS = 16
_NW = _NC * _NS
_CHUNK = 80

_F32 = jnp.float32
_HIGH = jax.lax.Precision.DEFAULT


def _dot(a, b):
    return jnp.dot(a, b, precision=_HIGH, preferred_element_type=_F32)


def _sc_degree(ei2, n_nodes):
    per_w = ei2.shape[2]
    mesh = plsc.VectorSubcoreMesh(
        core_axis_name="c", subcore_axis_name="s",
        num_cores=_NC, num_subcores=_NS)

    @functools.partial(
        pl.kernel,
        out_type=jax.ShapeDtypeStruct((_NW, n_nodes), _F32),
        mesh=mesh,
        scratch_types=[
            pltpu.VMEM((per_w,), jnp.int32),
            pltpu.VMEM((n_nodes,), _F32),
        ],
        compiler_params=pltpu.CompilerParams(needs_layout_passes=False),
    )
    def k(ei_hbm, out_hbm, dst_v, deg_v):
        wid = lax.axis_index("s") * _NC + lax.axis_index("c")
        pltpu.sync_copy(ei_hbm.at[1, wid], dst_v)

        def zero_body(i, _):
            deg_v[pl.ds(i * 16, 16)] = jnp.zeros((16,), _F32)
            return 0

        lax.fori_loop(0, n_nodes // 16, zero_body, 0)
        ones = jnp.ones((16,), _F32)

        def body(i, _):
            idx = dst_v[pl.ds(i * 16, 16)]
            plsc.addupdate_scatter(deg_v, [idx], ones)
            return 0

        lax.fori_loop(0, per_w // 16, body, 0)
        pltpu.sync_copy(deg_v, out_hbm.at[wid])

    return k(ei2)


def _sc_scatter(ei2, ei4, gs, zeros_tile):
    n_phase = len(gs)
    n, d = gs[0].shape
    per_w = ei2.shape[2]
    chunks_per_w = ei4.shape[2]
    n_pad = zeros_tile.shape[0] * _NS
    rows_per_tile = n_pad // _NS
    mesh = plsc.VectorSubcoreMesh(
        core_axis_name="c", subcore_axis_name="s",
        num_cores=_NC, num_subcores=_NS)

    assert chunks_per_w % 2 == 1

    @functools.partial(
        pl.kernel,
        out_type=jax.ShapeDtypeStruct((n_phase, _NC, n_pad, d), _F32),
        mesh=mesh,
        scratch_types=[
            pltpu.VMEM((per_w,), jnp.int32),
            pltpu.VMEM((chunks_per_w, _CHUNK), jnp.int32),
            [pltpu.VMEM((_CHUNK, d), _F32) for _ in range(2)],
            pltpu.VMEM_SHARED((n_pad, d), _F32),
            [pltpu.SemaphoreType.DMA for _ in range(2)],
        ],
        compiler_params=pltpu.CompilerParams(needs_layout_passes=False),
    )
    def k(ei_hbm, ei4_hbm, *refs):
        g_hbms = refs[:n_phase]
        zero_hbm, out_hbm, src_v, dst_v, rows, acc, semg = refs[n_phase:]
        cid = lax.axis_index("c")
        sid = lax.axis_index("s")
        wid = sid * _NC + cid
        my_rows = pl.ds(sid * rows_per_tile, rows_per_tile)
        pltpu.sync_copy(zero_hbm, acc.at[my_rows])
        pltpu.sync_copy(ei_hbm.at[0, wid], src_v)
        pltpu.sync_copy(ei4_hbm.at[1, wid], dst_v)
        plsc.subcore_barrier()

        def run_phase(g_hbm):
            def gather_start(j, b):
                pltpu.async_copy(
                    g_hbm.at[src_v.at[pl.ds(j * _CHUNK, _CHUNK)]], rows[b],
                    semg[b])

            def gather_wait(j, b):
                pltpu.make_async_copy(
                    g_hbm.at[src_v.at[pl.ds(j * _CHUNK, _CHUNK)]], rows[b],
                    semg[b]).wait()

            def scat(j, b):
                pltpu.sync_copy(rows[b], acc.at[dst_v.at[j]], add=True)

            last = chunks_per_w - 1
            gather_start(0, 0)

            def body(jj, _):
                j = 2 * jj
                gather_start(j + 1, 1)
                gather_wait(j, 0)
                scat(j, 0)
                gather_start(j + 2, 0)
                gather_wait(j + 1, 1)
                scat(j + 1, 1)
                return 0

            lax.fori_loop(0, chunks_per_w // 2, body, 0)
            gather_wait(last, 0)
            scat(last, 0)

        for p, g_hbm in enumerate(g_hbms):
            if p > 0:
                pltpu.sync_copy(zero_hbm, acc.at[my_rows])
                plsc.subcore_barrier()
            run_phase(g_hbm)
            plsc.subcore_barrier()
            pltpu.sync_copy(acc.at[my_rows], out_hbm.at[p, cid, my_rows])

    return k(ei2, ei4, *gs, zeros_tile)


_ROWS = 2000


def _tc_prep(deg_parts, x):
    n, d = x.shape

    def body(deg_ref, x_ref, dis_ref, g_ref):
        deg = jnp.sum(deg_ref[...], axis=0)
        dis = jnp.where(deg > 0, lax.rsqrt(jnp.maximum(deg, 1e-12)), 0.0)
        dis = dis[:, None]
        dis_ref[...] = dis
        g_ref[...] = dis * x_ref[...]

    return pl.pallas_call(
        body,
        out_shape=[
            jax.ShapeDtypeStruct((n, 1), _F32),
            jax.ShapeDtypeStruct((n, d), _F32),
        ],
    )(deg_parts, x)


def _tc_combine_scale(parts, dis, squared):
    _, n, d = parts.shape
    grid = (n // _ROWS,)

    def body(p_ref, dis_ref, o_ref):
        s = dis_ref[...]
        if squared:
            s = s * s
        o_ref[...] = s * (p_ref[0] + p_ref[1])

    return pl.pallas_call(
        body,
        grid=grid,
        in_specs=[
            pl.BlockSpec((2, _ROWS, d), lambda i: (0, i, 0)),
            pl.BlockSpec((_ROWS, 1), lambda i: (i, 0)),
        ],
        out_specs=pl.BlockSpec((_ROWS, d), lambda i: (i, 0)),
        out_shape=jax.ShapeDtypeStruct((n, d), _F32),
    )(parts, dis)


def _tc_layer(u_parts, v_parts, dis, x, w10, w11, w12, w20, w21, w22):
    n, d = x.shape
    grid = (n // _ROWS,)

    def body(u_ref, v_ref, dis_ref, x_ref,
             w10_ref, w11_ref, w12_ref, w20_ref, w21_ref, w22_ref,
             q0_ref, m1_ref, m2_ref):
        dis_b = dis_ref[...]
        u = dis_b * (u_ref[0] + u_ref[1])
        v = dis_b * (v_ref[0] + v_ref[1])
        h0 = _dot(x_ref[...], w10_ref[...])
        h1 = _dot(u, w11_ref[...])
        h2 = _dot(v, w12_ref[...])
        h = jax.nn.relu(jnp.concatenate([h0, h1, h2], axis=-1))
        q0_ref[...] = _dot(h, w20_ref[...])
        m1_ref[...] = dis_b * _dot(h, w21_ref[...])
        m2_ref[...] = dis_b * _dot(h, w22_ref[...])

    wspec = lambda shape: pl.BlockSpec(shape, lambda i: (0, 0))
    return pl.pallas_call(
        body,
        grid=grid,
        in_specs=[
            pl.BlockSpec((2, _ROWS, d), lambda i: (0, i, 0)),
            pl.BlockSpec((2, _ROWS, d), lambda i: (0, i, 0)),
            pl.BlockSpec((_ROWS, 1), lambda i: (i, 0)),
            pl.BlockSpec((_ROWS, d), lambda i: (i, 0)),
            wspec(w10.shape), wspec(w11.shape), wspec(w12.shape),
            wspec(w20.shape), wspec(w21.shape), wspec(w22.shape),
        ],
        out_specs=[
            pl.BlockSpec((_ROWS, d), lambda i: (i, 0)),
            pl.BlockSpec((_ROWS, d), lambda i: (i, 0)),
            pl.BlockSpec((_ROWS, d), lambda i: (i, 0)),
        ],
        out_shape=[
            jax.ShapeDtypeStruct((n, d), _F32),
            jax.ShapeDtypeStruct((n, d), _F32),
            jax.ShapeDtypeStruct((n, d), _F32),
        ],
    )(u_parts, v_parts, dis, x, w10, w11, w12, w20, w21, w22)


def _tc_final(q0, a_parts, c_parts, dis):
    n, d = q0.shape
    grid = (n // _ROWS,)

    def body(q0_ref, a_ref, c_ref, dis_ref, o_ref):
        dis_b = dis_ref[...]
        q1 = dis_b * (a_ref[0] + a_ref[1])
        q2 = dis_b * (c_ref[0] + c_ref[1])
        o_ref[...] = jnp.concatenate([q0_ref[...], q1, q2], axis=-1)

    return pl.pallas_call(
        body,
        grid=grid,
        in_specs=[
            pl.BlockSpec((_ROWS, d), lambda i: (i, 0)),
            pl.BlockSpec((2, _ROWS, d), lambda i: (0, i, 0)),
            pl.BlockSpec((2, _ROWS, d), lambda i: (0, i, 0)),
            pl.BlockSpec((_ROWS, 1), lambda i: (i, 0)),
        ],
        out_specs=pl.BlockSpec((_ROWS, 3 * d), lambda i: (i, 0)),
        out_shape=jax.ShapeDtypeStruct((n, 3 * d), _F32),
    )(q0, a_parts, c_parts, dis)


def kernel(x, edge_index, W1_0, W1_1, W1_2, W2_0, W2_1, W2_2):
    n, d = x.shape
    e = edge_index.shape[1]
    ei = edge_index.astype(jnp.int32)
    chunks_per_w = e // (_NW * _CHUNK)
    ei2 = ei.reshape(2, _NW, e // _NW)
    ei4 = ei.reshape(2, _NW, chunks_per_w, _CHUNK)
    rpt = -(-n // (8 * _NS)) * 8
    zeros_tile = jnp.zeros((rpt, d), _F32)

    deg_parts = _sc_degree(ei2, n)
    dis, g = _tc_prep(deg_parts, x)

    u_parts = _sc_scatter(ei2, ei4, [g], zeros_tile)[0]
    g2 = _tc_combine_scale(u_parts, dis, squared=True)
    v_parts = _sc_scatter(ei2, ei4, [g2], zeros_tile)[0]

    q0, m1, m2 = _tc_layer(u_parts, v_parts, dis, x,
                           W1_0, W1_1, W1_2, W2_0, W2_1, W2_2)

    a_parts = _sc_scatter(ei2, ei4, [m1], zeros_tile)[0]
    b_parts = _sc_scatter(ei2, ei4, [m2], zeros_tile)[0]
    t = _tc_combine_scale(b_parts, dis, squared=True)
    c_parts = _sc_scatter(ei2, ei4, [t], zeros_tile)[0]

    return _tc_final(q0, a_parts, c_parts, dis)


# speedup vs baseline: 1.3818x; 1.0329x over previous
import functools

import jax
import jax.numpy as jnp
from jax import lax
from jax.experimental import pallas as pl
from jax.experimental.pallas import tpu as pltpu
from jax.experimental.pallas import tpu_sc as plsc

_NC = 2
_NS = 16
_NW = _NC * _NS
_CHUNK = 80

_F32 = jnp.float32
_HIGH = jax.lax.Precision.DEFAULT


def _dot(a, b):
    return jnp.dot(a, b, precision=_HIGH, preferred_element_type=_F32)


def _sc_degree(ei2, n_nodes):
    per_w = ei2.shape[2]
    mesh = plsc.VectorSubcoreMesh(
        core_axis_name="c", subcore_axis_name="s",
        num_cores=_NC, num_subcores=_NS)

    @functools.partial(
        pl.kernel,
        out_type=jax.ShapeDtypeStruct((_NW, n_nodes), _F32),
        mesh=mesh,
        scratch_types=[
            pltpu.VMEM((per_w,), jnp.int32),
            pltpu.VMEM((n_nodes,), _F32),
        ],
        compiler_params=pltpu.CompilerParams(needs_layout_passes=False),
    )
    def k(ei_hbm, out_hbm, dst_v, deg_v):
        wid = lax.axis_index("s") * _NC + lax.axis_index("c")
        pltpu.sync_copy(ei_hbm.at[1, wid], dst_v)

        def zero_body(i, _):
            deg_v[pl.ds(i * 16, 16)] = jnp.zeros((16,), _F32)
            return 0

        lax.fori_loop(0, n_nodes // 16, zero_body, 0)
        ones = jnp.ones((16,), _F32)

        def body(i, _):
            idx = dst_v[pl.ds(i * 16, 16)]
            plsc.addupdate_scatter(deg_v, [idx], ones)
            return 0

        lax.fori_loop(0, per_w // 16, body, 0)
        pltpu.sync_copy(deg_v, out_hbm.at[wid])

    return k(ei2)


def _sc_scatter(ei2, ei4, gs, zeros_tile):
    n_phase = len(gs)
    n, d = gs[0].shape
    per_w = ei2.shape[2]
    chunks_per_w = ei4.shape[2]
    n_pad = zeros_tile.shape[0] * _NS
    rows_per_tile = n_pad // _NS
    mesh = plsc.VectorSubcoreMesh(
        core_axis_name="c", subcore_axis_name="s",
        num_cores=_NC, num_subcores=_NS)

    assert chunks_per_w % 2 == 1

    @functools.partial(
        pl.kernel,
        out_type=jax.ShapeDtypeStruct((n_phase, _NC, n_pad, d), _F32),
        mesh=mesh,
        scratch_types=[
            pltpu.VMEM((per_w,), jnp.int32),
            pltpu.VMEM((chunks_per_w, _CHUNK), jnp.int32),
            [pltpu.VMEM((_CHUNK, d), _F32) for _ in range(2)],
            pltpu.VMEM_SHARED((n_pad, d), _F32),
            [pltpu.SemaphoreType.DMA for _ in range(2)],
        ],
        compiler_params=pltpu.CompilerParams(needs_layout_passes=False),
    )
    def k(ei_hbm, ei4_hbm, *refs):
        g_hbms = refs[:n_phase]
        zero_hbm, out_hbm, src_v, dst_v, rows, acc, semg = refs[n_phase:]
        del zero_hbm
        cid = lax.axis_index("c")
        sid = lax.axis_index("s")
        wid = sid * _NC + cid
        base = sid * rows_per_tile
        my_rows = pl.ds(base, rows_per_tile)

        def zero_acc():
            def zrow(r, _):
                def zcol(c, _):
                    rows[0][r, pl.ds(c * 16, 16)] = jnp.zeros((16,), _F32)
                    return 0
                lax.fori_loop(0, d // 16, zcol, 0)
                return 0

            lax.fori_loop(0, _CHUNK, zrow, 0)
            n_full = rows_per_tile // _CHUNK
            for kk in range(n_full):
                pltpu.sync_copy(rows[0], acc.at[pl.ds(base + kk * _CHUNK, _CHUNK)])
            rem = rows_per_tile - n_full * _CHUNK
            if rem:
                pltpu.sync_copy(rows[0].at[pl.ds(0, rem)],
                                acc.at[pl.ds(base + n_full * _CHUNK, rem)])

        zero_acc()
        pltpu.sync_copy(ei_hbm.at[0, wid], src_v)
        pltpu.sync_copy(ei4_hbm.at[1, wid], dst_v)
        plsc.subcore_barrier()

        def run_phase(g_hbm):
            def gather_start(j, b):
                pltpu.async_copy(
                    g_hbm.at[src_v.at[pl.ds(j * _CHUNK, _CHUNK)]], rows[b],
                    semg[b])

            def gather_wait(j, b):
                pltpu.make_async_copy(
                    g_hbm.at[src_v.at[pl.ds(j * _CHUNK, _CHUNK)]], rows[b],
                    semg[b]).wait()

            def scat(j, b):
                pltpu.sync_copy(rows[b], acc.at[dst_v.at[j]], add=True)

            last = chunks_per_w - 1
            gather_start(0, 0)

            def body(jj, _):
                j = 2 * jj
                gather_start(j + 1, 1)
                gather_wait(j, 0)
                scat(j, 0)
                gather_start(j + 2, 0)
                gather_wait(j + 1, 1)
                scat(j + 1, 1)
                return 0

            lax.fori_loop(0, chunks_per_w // 2, body, 0)
            gather_wait(last, 0)
            scat(last, 0)

        for p, g_hbm in enumerate(g_hbms):
            if p > 0:
                zero_acc()
                plsc.subcore_barrier()
            run_phase(g_hbm)
            plsc.subcore_barrier()
            pltpu.sync_copy(acc.at[my_rows], out_hbm.at[p, cid, my_rows])

    return k(ei2, ei4, *gs, zeros_tile)


_ROWS = 2000


def _tc_prep(deg_parts, x):
    n, d = x.shape

    def body(deg_ref, x_ref, dis_ref, g_ref):
        deg = jnp.sum(deg_ref[...], axis=0)
        dis = jnp.where(deg > 0, lax.rsqrt(jnp.maximum(deg, 1e-12)), 0.0)
        dis = dis[:, None]
        dis_ref[...] = dis
        g_ref[...] = dis * x_ref[...]

    return pl.pallas_call(
        body,
        out_shape=[
            jax.ShapeDtypeStruct((n, 1), _F32),
            jax.ShapeDtypeStruct((n, d), _F32),
        ],
    )(deg_parts, x)


def _tc_combine_scale(parts, dis, squared):
    _, n, d = parts.shape
    grid = (n // _ROWS,)

    def body(p_ref, dis_ref, o_ref):
        s = dis_ref[...]
        if squared:
            s = s * s
        o_ref[...] = s * (p_ref[0] + p_ref[1])

    return pl.pallas_call(
        body,
        grid=grid,
        in_specs=[
            pl.BlockSpec((2, _ROWS, d), lambda i: (0, i, 0)),
            pl.BlockSpec((_ROWS, 1), lambda i: (i, 0)),
        ],
        out_specs=pl.BlockSpec((_ROWS, d), lambda i: (i, 0)),
        out_shape=jax.ShapeDtypeStruct((n, d), _F32),
    )(parts, dis)


def _tc_layer(u_parts, v_parts, dis, x, w10, w11, w12, w20, w21, w22):
    n, d = x.shape
    grid = (n // _ROWS,)

    def body(u_ref, v_ref, dis_ref, x_ref,
             w10_ref, w11_ref, w12_ref, w20_ref, w21_ref, w22_ref,
             q0_ref, m1_ref, m2_ref):
        dis_b = dis_ref[...]
        u = dis_b * (u_ref[0] + u_ref[1])
        v = dis_b * (v_ref[0] + v_ref[1])
        h0 = _dot(x_ref[...], w10_ref[...])
        h1 = _dot(u, w11_ref[...])
        h2 = _dot(v, w12_ref[...])
        h = jax.nn.relu(jnp.concatenate([h0, h1, h2], axis=-1))
        q0_ref[...] = _dot(h, w20_ref[...])
        m1_ref[...] = dis_b * _dot(h, w21_ref[...])
        m2_ref[...] = dis_b * _dot(h, w22_ref[...])

    wspec = lambda shape: pl.BlockSpec(shape, lambda i: (0, 0))
    return pl.pallas_call(
        body,
        grid=grid,
        in_specs=[
            pl.BlockSpec((2, _ROWS, d), lambda i: (0, i, 0)),
            pl.BlockSpec((2, _ROWS, d), lambda i: (0, i, 0)),
            pl.BlockSpec((_ROWS, 1), lambda i: (i, 0)),
            pl.BlockSpec((_ROWS, d), lambda i: (i, 0)),
            wspec(w10.shape), wspec(w11.shape), wspec(w12.shape),
            wspec(w20.shape), wspec(w21.shape), wspec(w22.shape),
        ],
        out_specs=[
            pl.BlockSpec((_ROWS, d), lambda i: (i, 0)),
            pl.BlockSpec((_ROWS, d), lambda i: (i, 0)),
            pl.BlockSpec((_ROWS, d), lambda i: (i, 0)),
        ],
        out_shape=[
            jax.ShapeDtypeStruct((n, d), _F32),
            jax.ShapeDtypeStruct((n, d), _F32),
            jax.ShapeDtypeStruct((n, d), _F32),
        ],
    )(u_parts, v_parts, dis, x, w10, w11, w12, w20, w21, w22)


def _tc_final(q0, a_parts, c_parts, dis):
    n, d = q0.shape
    grid = (n // _ROWS,)

    def body(q0_ref, a_ref, c_ref, dis_ref, o_ref):
        dis_b = dis_ref[...]
        q1 = dis_b * (a_ref[0] + a_ref[1])
        q2 = dis_b * (c_ref[0] + c_ref[1])
        o_ref[...] = jnp.concatenate([q0_ref[...], q1, q2], axis=-1)

    return pl.pallas_call(
        body,
        grid=grid,
        in_specs=[
            pl.BlockSpec((_ROWS, d), lambda i: (i, 0)),
            pl.BlockSpec((2, _ROWS, d), lambda i: (0, i, 0)),
            pl.BlockSpec((2, _ROWS, d), lambda i: (0, i, 0)),
            pl.BlockSpec((_ROWS, 1), lambda i: (i, 0)),
        ],
        out_specs=pl.BlockSpec((_ROWS, 3 * d), lambda i: (i, 0)),
        out_shape=jax.ShapeDtypeStruct((n, 3 * d), _F32),
    )(q0, a_parts, c_parts, dis)


def kernel(x, edge_index, W1_0, W1_1, W1_2, W2_0, W2_1, W2_2):
    n, d = x.shape
    e = edge_index.shape[1]
    ei = edge_index.astype(jnp.int32)
    chunks_per_w = e // (_NW * _CHUNK)
    ei2 = ei.reshape(2, _NW, e // _NW)
    ei4 = ei.reshape(2, _NW, chunks_per_w, _CHUNK)
    rpt = -(-n // (8 * _NS)) * 8
    zeros_tile = jnp.zeros((rpt, d), _F32)

    deg_parts = _sc_degree(ei2, n)
    dis, g = _tc_prep(deg_parts, x)

    u_parts = _sc_scatter(ei2, ei4, [g], zeros_tile)[0]
    g2 = _tc_combine_scale(u_parts, dis, squared=True)
    v_parts = _sc_scatter(ei2, ei4, [g2], zeros_tile)[0]

    q0, m1, m2 = _tc_layer(u_parts, v_parts, dis, x,
                           W1_0, W1_1, W1_2, W2_0, W2_1, W2_2)

    a_parts = _sc_scatter(ei2, ei4, [m1], zeros_tile)[0]
    b_parts = _sc_scatter(ei2, ei4, [m2], zeros_tile)[0]
    t = _tc_combine_scale(b_parts, dis, squared=True)
    c_parts = _sc_scatter(ei2, ei4, [t], zeros_tile)[0]

    return _tc_final(q0, a_parts, c_parts, dis)


# speedup vs baseline: 1.3835x; 1.0012x over previous
import functools

import jax
import jax.numpy as jnp
from jax import lax
from jax.experimental import pallas as pl
from jax.experimental.pallas import tpu as pltpu
from jax.experimental.pallas import tpu_sc as plsc

_NC = 2
_NS = 16
_NW = _NC * _NS
_CHUNK = 80

_F32 = jnp.float32
_HIGH = jax.lax.Precision.DEFAULT


def _dot(a, b):
    return jnp.dot(a, b, precision=_HIGH, preferred_element_type=_F32)


def _sc_degree(ei2, n_nodes):
    per_w = ei2.shape[2]
    mesh = plsc.VectorSubcoreMesh(
        core_axis_name="c", subcore_axis_name="s",
        num_cores=_NC, num_subcores=_NS)

    @functools.partial(
        pl.kernel,
        out_type=jax.ShapeDtypeStruct((_NW, n_nodes), _F32),
        mesh=mesh,
        scratch_types=[
            pltpu.VMEM((per_w,), jnp.int32),
            pltpu.VMEM((n_nodes,), _F32),
        ],
        compiler_params=pltpu.CompilerParams(needs_layout_passes=False),
    )
    def k(ei_hbm, out_hbm, dst_v, deg_v):
        wid = lax.axis_index("s") * _NC + lax.axis_index("c")
        pltpu.sync_copy(ei_hbm.at[1, wid], dst_v)

        def zero_body(i, _):
            deg_v[pl.ds(i * 16, 16)] = jnp.zeros((16,), _F32)
            return 0

        lax.fori_loop(0, n_nodes // 16, zero_body, 0)
        ones = jnp.ones((16,), _F32)

        def body(i, _):
            idx = dst_v[pl.ds(i * 16, 16)]
            plsc.addupdate_scatter(deg_v, [idx], ones)
            return 0

        lax.fori_loop(0, per_w // 16, body, 0)
        pltpu.sync_copy(deg_v, out_hbm.at[wid])

    return k(ei2)


def _sc_scatter(ei2, ei4, gs, n_pad):
    n_phase = len(gs)
    n, d = gs[0].shape
    per_w = ei2.shape[2]
    chunks_per_w = ei4.shape[2]
    rows_per_tile = n_pad // _NS
    mesh = plsc.VectorSubcoreMesh(
        core_axis_name="c", subcore_axis_name="s",
        num_cores=_NC, num_subcores=_NS)

    assert chunks_per_w % 2 == 1

    @functools.partial(
        pl.kernel,
        out_type=jax.ShapeDtypeStruct((n_phase, _NC, n_pad, d), _F32),
        mesh=mesh,
        scratch_types=[
            pltpu.VMEM((per_w,), jnp.int32),
            pltpu.VMEM((chunks_per_w, _CHUNK), jnp.int32),
            [pltpu.VMEM((_CHUNK, d), _F32) for _ in range(2)],
            pltpu.VMEM_SHARED((n_pad, d), _F32),
            [pltpu.SemaphoreType.DMA for _ in range(2)],
        ],
        compiler_params=pltpu.CompilerParams(needs_layout_passes=False),
    )
    def k(ei_hbm, ei4_hbm, *refs):
        g_hbms = refs[:n_phase]
        out_hbm, src_v, dst_v, rows, acc, semg = refs[n_phase:]
        cid = lax.axis_index("c")
        sid = lax.axis_index("s")
        wid = sid * _NC + cid
        base = sid * rows_per_tile
        my_rows = pl.ds(base, rows_per_tile)

        def zero_acc():
            def zrow(r, _):
                def zcol(c, _):
                    rows[0][r, pl.ds(c * 16, 16)] = jnp.zeros((16,), _F32)
                    return 0
                lax.fori_loop(0, d // 16, zcol, 0)
                return 0

            lax.fori_loop(0, _CHUNK, zrow, 0)
            n_full = rows_per_tile // _CHUNK
            for kk in range(n_full):
                pltpu.sync_copy(rows[0], acc.at[pl.ds(base + kk * _CHUNK, _CHUNK)])
            rem = rows_per_tile - n_full * _CHUNK
            if rem:
                pltpu.sync_copy(rows[0].at[pl.ds(0, rem)],
                                acc.at[pl.ds(base + n_full * _CHUNK, rem)])

        zero_acc()
        pltpu.sync_copy(ei_hbm.at[0, wid], src_v)
        pltpu.sync_copy(ei4_hbm.at[1, wid], dst_v)
        plsc.subcore_barrier()

        def run_phase(g_hbm):
            def gather_start(j, b):
                pltpu.async_copy(
                    g_hbm.at[src_v.at[pl.ds(j * _CHUNK, _CHUNK)]], rows[b],
                    semg[b])

            def gather_wait(j, b):
                pltpu.make_async_copy(
                    g_hbm.at[src_v.at[pl.ds(j * _CHUNK, _CHUNK)]], rows[b],
                    semg[b]).wait()

            def scat(j, b):
                pltpu.sync_copy(rows[b], acc.at[dst_v.at[j]], add=True)

            last = chunks_per_w - 1
            gather_start(0, 0)

            def body(jj, _):
                j = 2 * jj
                gather_start(j + 1, 1)
                gather_wait(j, 0)
                scat(j, 0)
                gather_start(j + 2, 0)
                gather_wait(j + 1, 1)
                scat(j + 1, 1)
                return 0

            lax.fori_loop(0, chunks_per_w // 2, body, 0)
            gather_wait(last, 0)
            scat(last, 0)

        for p, g_hbm in enumerate(g_hbms):
            if p > 0:
                zero_acc()
                plsc.subcore_barrier()
            run_phase(g_hbm)
            plsc.subcore_barrier()
            pltpu.sync_copy(acc.at[my_rows], out_hbm.at[p, cid, my_rows])

    return k(ei2, ei4, *gs)


_ROWS = 2000


def _tc_prep(deg_parts, x):
    n, d = x.shape

    def body(deg_ref, x_ref, dis_ref, g_ref):
        deg = jnp.sum(deg_ref[...], axis=0)
        dis = jnp.where(deg > 0, lax.rsqrt(jnp.maximum(deg, 1e-12)), 0.0)
        dis = dis[:, None]
        dis_ref[...] = dis
        g_ref[...] = dis * x_ref[...]

    return pl.pallas_call(
        body,
        out_shape=[
            jax.ShapeDtypeStruct((n, 1), _F32),
            jax.ShapeDtypeStruct((n, d), _F32),
        ],
    )(deg_parts, x)


def _tc_combine_scale(parts, dis, squared):
    _, n, d = parts.shape
    grid = (n // _ROWS,)

    def body(p_ref, dis_ref, o_ref):
        s = dis_ref[...]
        if squared:
            s = s * s
        o_ref[...] = s * (p_ref[0] + p_ref[1])

    return pl.pallas_call(
        body,
        grid=grid,
        in_specs=[
            pl.BlockSpec((2, _ROWS, d), lambda i: (0, i, 0)),
            pl.BlockSpec((_ROWS, 1), lambda i: (i, 0)),
        ],
        out_specs=pl.BlockSpec((_ROWS, d), lambda i: (i, 0)),
        out_shape=jax.ShapeDtypeStruct((n, d), _F32),
    )(parts, dis)


def _tc_layer(u_parts, v_parts, dis, x, w10, w11, w12, w20, w21, w22):
    n, d = x.shape
    grid = (n // _ROWS,)

    def body(u_ref, v_ref, dis_ref, x_ref,
             w10_ref, w11_ref, w12_ref, w20_ref, w21_ref, w22_ref,
             q0_ref, m1_ref, m2_ref):
        dis_b = dis_ref[...]
        u = dis_b * (u_ref[0] + u_ref[1])
        v = dis_b * (v_ref[0] + v_ref[1])
        h0 = _dot(x_ref[...], w10_ref[...])
        h1 = _dot(u, w11_ref[...])
        h2 = _dot(v, w12_ref[...])
        h = jax.nn.relu(jnp.concatenate([h0, h1, h2], axis=-1))
        q0_ref[...] = _dot(h, w20_ref[...])
        m1_ref[...] = dis_b * _dot(h, w21_ref[...])
        m2_ref[...] = dis_b * _dot(h, w22_ref[...])

    wspec = lambda shape: pl.BlockSpec(shape, lambda i: (0, 0))
    return pl.pallas_call(
        body,
        grid=grid,
        in_specs=[
            pl.BlockSpec((2, _ROWS, d), lambda i: (0, i, 0)),
            pl.BlockSpec((2, _ROWS, d), lambda i: (0, i, 0)),
            pl.BlockSpec((_ROWS, 1), lambda i: (i, 0)),
            pl.BlockSpec((_ROWS, d), lambda i: (i, 0)),
            wspec(w10.shape), wspec(w11.shape), wspec(w12.shape),
            wspec(w20.shape), wspec(w21.shape), wspec(w22.shape),
        ],
        out_specs=[
            pl.BlockSpec((_ROWS, d), lambda i: (i, 0)),
            pl.BlockSpec((_ROWS, d), lambda i: (i, 0)),
            pl.BlockSpec((_ROWS, d), lambda i: (i, 0)),
        ],
        out_shape=[
            jax.ShapeDtypeStruct((n, d), _F32),
            jax.ShapeDtypeStruct((n, d), _F32),
            jax.ShapeDtypeStruct((n, d), _F32),
        ],
    )(u_parts, v_parts, dis, x, w10, w11, w12, w20, w21, w22)


def _tc_final(q0, a_parts, c_parts, dis):
    n, d = q0.shape
    grid = (n // _ROWS,)

    def body(q0_ref, a_ref, c_ref, dis_ref, o_ref):
        dis_b = dis_ref[...]
        q1 = dis_b * (a_ref[0] + a_ref[1])
        q2 = dis_b * (c_ref[0] + c_ref[1])
        o_ref[...] = jnp.concatenate([q0_ref[...], q1, q2], axis=-1)

    return pl.pallas_call(
        body,
        grid=grid,
        in_specs=[
            pl.BlockSpec((_ROWS, d), lambda i: (i, 0)),
            pl.BlockSpec((2, _ROWS, d), lambda i: (0, i, 0)),
            pl.BlockSpec((2, _ROWS, d), lambda i: (0, i, 0)),
            pl.BlockSpec((_ROWS, 1), lambda i: (i, 0)),
        ],
        out_specs=pl.BlockSpec((_ROWS, 3 * d), lambda i: (i, 0)),
        out_shape=jax.ShapeDtypeStruct((n, 3 * d), _F32),
    )(q0, a_parts, c_parts, dis)


def kernel(x, edge_index, W1_0, W1_1, W1_2, W2_0, W2_1, W2_2):
    n, d = x.shape
    e = edge_index.shape[1]
    ei = edge_index.astype(jnp.int32)
    chunks_per_w = e // (_NW * _CHUNK)
    ei2 = ei.reshape(2, _NW, e // _NW)
    ei4 = ei.reshape(2, _NW, chunks_per_w, _CHUNK)
    n_pad = -(-n // (8 * _NS)) * 8 * _NS

    deg_parts = _sc_degree(ei2, n)
    dis, g = _tc_prep(deg_parts, x)

    u_parts = _sc_scatter(ei2, ei4, [g], n_pad)[0]
    g2 = _tc_combine_scale(u_parts, dis, squared=True)
    v_parts = _sc_scatter(ei2, ei4, [g2], n_pad)[0]

    q0, m1, m2 = _tc_layer(u_parts, v_parts, dis, x,
                           W1_0, W1_1, W1_2, W2_0, W2_1, W2_2)

    a_parts = _sc_scatter(ei2, ei4, [m1], n_pad)[0]
    b_parts = _sc_scatter(ei2, ei4, [m2], n_pad)[0]
    t = _tc_combine_scale(b_parts, dis, squared=True)
    c_parts = _sc_scatter(ei2, ei4, [t], n_pad)[0]

    return _tc_final(q0, a_parts, c_parts, dis)
